# baseline (speedup 1.0000x reference)
import functools

import jax
import jax.numpy as jnp
from jax import lax
from jax.experimental import pallas as pl
from jax.experimental.pallas import tpu as pltpu
from jax.experimental.pallas import tpu_sc as plsc

TYPE_VOCAB = 2
D = 1024
N_ROWS = 4 * 8192

_info = plsc.get_sparse_core_info()
NC, NS = _info.num_cores, _info.num_subcores
NW = NC * NS
ROWS_PER_W = N_ROWS // NW
SUPER_ROWS = 64
NSUPER = ROWS_PER_W // SUPER_ROWS


def _body(ids_hbm, table_hbm, out_hbm, idx_v, tbl_v, drain_v, sem, semb):
    cid = lax.axis_index("c")
    sid = lax.axis_index("s")
    wid = sid * NC + cid
    base = wid * ROWS_PER_W

    h_tbl = pltpu.async_copy(table_hbm, tbl_v, semb)
    h_idx = pltpu.async_copy(ids_hbm.at[pl.ds(base, ROWS_PER_W)], idx_v, sem)
    h_tbl.wait()
    h_idx.wait()

    def super_body(sg, carry):
        for g in range(SUPER_ROWS // 16):
            gbase = sg * SUPER_ROWS + g * 16
            ivec = idx_v[pl.ds(gbase, 16)]
            for r in range(16):
                s = lax.squeeze(lax.slice(ivec, (r,), (r + 1,)), (0,))
                pltpu.async_copy(
                    tbl_v.at[pl.ds(s, 1)],
                    out_hbm.at[pl.ds(base + gbase + r, 1)], sem)

        @pl.when(sg >= 3)
        def _drain():
            pltpu.make_async_copy(
                out_hbm.at[pl.ds(base, SUPER_ROWS)], drain_v, sem).wait()

        return carry

    lax.fori_loop(0, NSUPER, super_body, 0)
    for _ in range(3):
        pltpu.make_async_copy(
            out_hbm.at[pl.ds(base, SUPER_ROWS)], drain_v, sem).wait()


@jax.jit
def _run(ids_flat, table):
    mesh = plsc.VectorSubcoreMesh(core_axis_name="c", subcore_axis_name="s")
    f = functools.partial(
        pl.kernel,
        mesh=mesh,
        out_type=jax.ShapeDtypeStruct((N_ROWS, D), jnp.float32),
        scratch_types=[
            pltpu.VMEM((ROWS_PER_W,), jnp.int32),
            pltpu.VMEM((TYPE_VOCAB, D), jnp.float32),
            pltpu.VMEM((SUPER_ROWS, D), jnp.float32),
            pltpu.SemaphoreType.DMA,
            pltpu.SemaphoreType.DMA,
        ],
    )(_body)
    return f(ids_flat, table)


def kernel(token_type_ids, segment_embedding_weight):
    b, s = token_type_ids.shape
    ids_flat = token_type_ids.reshape(-1).astype(jnp.int32)
    out = _run(ids_flat, segment_embedding_weight)
    return out.reshape(b, s, D)


# speedup vs baseline: 1.0140x; 1.0140x over previous
import functools

import jax
import jax.numpy as jnp
from jax import lax
from jax.experimental import pallas as pl
from jax.experimental.pallas import tpu as pltpu
from jax.experimental.pallas import tpu_sc as plsc

TYPE_VOCAB = 2
D = 1024
N_ROWS = 4 * 8192

_info = plsc.get_sparse_core_info()
NC, NS = _info.num_cores, _info.num_subcores
NW = NC * NS
ROWS_PER_W = N_ROWS // NW
SUPER_ROWS = 64
NSUPER = ROWS_PER_W // SUPER_ROWS


def _body(ids_hbm, table_hbm, out_hbm, idx_v, tbl_v, drain_v, sem):
    cid = lax.axis_index("c")
    sid = lax.axis_index("s")
    wid = sid * NC + cid
    base = wid * ROWS_PER_W

    pltpu.sync_copy(table_hbm, tbl_v)
    pltpu.sync_copy(ids_hbm.at[pl.ds(base, ROWS_PER_W)], idx_v)

    def super_body(sg, carry):
        for g in range(SUPER_ROWS // 16):
            gbase = sg * SUPER_ROWS + g * 16
            ivec = idx_v[pl.ds(gbase, 16)]
            for r in range(16):
                s = lax.squeeze(lax.slice(ivec, (r,), (r + 1,)), (0,))
                pltpu.async_copy(
                    tbl_v.at[pl.ds(s, 1)],
                    out_hbm.at[pl.ds(base + gbase + r, 1)], sem)

        @pl.when(sg >= 3)
        def _drain():
            pltpu.make_async_copy(
                out_hbm.at[pl.ds(base, SUPER_ROWS)], drain_v, sem).wait()

        return carry

    lax.fori_loop(0, NSUPER, super_body, 0)
    for _ in range(3):
        pltpu.make_async_copy(
            out_hbm.at[pl.ds(base, SUPER_ROWS)], drain_v, sem).wait()


@jax.jit
def _run(ids_flat, table):
    mesh = plsc.VectorSubcoreMesh(core_axis_name="c", subcore_axis_name="s")
    f = functools.partial(
        pl.kernel,
        mesh=mesh,
        out_type=jax.ShapeDtypeStruct((N_ROWS, D), jnp.float32),
        scratch_types=[
            pltpu.VMEM((ROWS_PER_W,), jnp.int32),
            pltpu.VMEM((TYPE_VOCAB, D), jnp.float32),
            pltpu.VMEM((SUPER_ROWS, D), jnp.float32),
            pltpu.SemaphoreType.DMA,
        ],
    )(_body)
    return f(ids_flat, table)


def kernel(token_type_ids, segment_embedding_weight):
    b, s = token_type_ids.shape
    ids_flat = token_type_ids.reshape(-1).astype(jnp.int32)
    out = _run(ids_flat, segment_embedding_weight)
    return out.reshape(b, s, D)


# speedup vs baseline: 1.0162x; 1.0022x over previous
import functools

import jax
import jax.numpy as jnp
from jax import lax
from jax.experimental import pallas as pl
from jax.experimental.pallas import tpu as pltpu
from jax.experimental.pallas import tpu_sc as plsc

TYPE_VOCAB = 2
D = 1024
N_ROWS = 4 * 8192

_info = plsc.get_sparse_core_info()
NC, NS = _info.num_cores, _info.num_subcores
NW = NC * NS
ROWS_PER_W = N_ROWS // NW
SUPER_ROWS = 64
NSUPER = ROWS_PER_W // SUPER_ROWS


def _body(ids_hbm, table_hbm, out_hbm, idx_v, tbl_v, drain_v, sem):
    cid = lax.axis_index("c")
    sid = lax.axis_index("s")
    wid = sid * NC + cid
    base = wid * ROWS_PER_W

    pltpu.sync_copy(table_hbm, tbl_v)
    pltpu.sync_copy(ids_hbm.at[pl.ds(base, ROWS_PER_W)], idx_v)

    def super_body(sg, carry):
        for g in range(SUPER_ROWS // 16):
            gbase = sg * SUPER_ROWS + g * 16
            ivec = idx_v[pl.ds(gbase, 16)]
            for r in range(16):
                s = lax.squeeze(lax.slice(ivec, (r,), (r + 1,)), (0,))
                for h in range(2):
                    pltpu.async_copy(
                        tbl_v.at[pl.ds(s, 1), pl.ds(h * (D // 2), D // 2)],
                        out_hbm.at[pl.ds(base + gbase + r, 1),
                                   pl.ds(h * (D // 2), D // 2)], sem)

        @pl.when(sg >= 3)
        def _drain():
            pltpu.make_async_copy(
                out_hbm.at[pl.ds(base, SUPER_ROWS)], drain_v, sem).wait()

        return carry

    lax.fori_loop(0, NSUPER, super_body, 0)
    for _ in range(3):
        pltpu.make_async_copy(
            out_hbm.at[pl.ds(base, SUPER_ROWS)], drain_v, sem).wait()


@jax.jit
def _run(ids_flat, table):
    mesh = plsc.VectorSubcoreMesh(core_axis_name="c", subcore_axis_name="s")
    f = functools.partial(
        pl.kernel,
        mesh=mesh,
        out_type=jax.ShapeDtypeStruct((N_ROWS, D), jnp.float32),
        scratch_types=[
            pltpu.VMEM((ROWS_PER_W,), jnp.int32),
            pltpu.VMEM((TYPE_VOCAB, D), jnp.float32),
            pltpu.VMEM((SUPER_ROWS, D), jnp.float32),
            pltpu.SemaphoreType.DMA,
        ],
    )(_body)
    return f(ids_flat, table)


def kernel(token_type_ids, segment_embedding_weight):
    b, s = token_type_ids.shape
    ids_flat = token_type_ids.reshape(-1).astype(jnp.int32)
    out = _run(ids_flat, segment_embedding_weight)
    return out.reshape(b, s, D)
